# Initial kernel scaffold; baseline (speedup 1.0000x reference)
import jax
import jax.numpy as jnp
from jax.experimental import pallas as pl


def kernel(user_emb, user_emb_ego, dele_sim, add_sim, W_map, b_map, dele_indices, add_indices, batch_user):
    raise NotImplementedError("write your pallas kernel here")


# speedup vs baseline: 5.7016x; 5.7016x over previous
import functools

import jax
import jax.numpy as jnp
from jax import lax
from jax.experimental import pallas as pl
from jax.experimental.pallas import tpu as pltpu
from jax.experimental.pallas import tpu_sc as plsc

_N = 10000
_D = 256
_H = 128
_E = 160000
_B = 4096
_NT = 16
_CH = 128
_NCH = 80
_EPT = _CH * _NCH
_EP = _EPT * _NT
_N2 = 10240
_PADROW = 10001


def _sc_body(user2, ego2, drows, dcols, dsim, arows, acols, asim, batch,
             uu_o, emb_o, ego_o,
             acc, s_d, s_a, rows_t, vals_t, colb, gbuf, sbuf, rowbuf, zs,
             bidx):
    cid = lax.axis_index("c")
    sid = lax.axis_index("s")
    f32 = jnp.float32

    def _zrow(i, _):
        for k in range(8):
            rowbuf[i, pl.ds(16 * k, 16)] = jnp.zeros((16,), f32)
        return 0
    lax.fori_loop(0, _CH, _zrow, 0)

    def _zv(i, _):
        zs[pl.ds(16 * i, 16)] = jnp.zeros((16,), f32)
        return 0
    lax.fori_loop(0, 40, _zv, 0)

    abase = sid * 640
    def _zacc(j, _):
        pltpu.sync_copy(rowbuf, acc.at[pl.ds(abase + j * _CH, _CH)])
        return 0
    lax.fori_loop(0, 5, _zacc, 0)
    pltpu.sync_copy(zs, s_d.at[pl.ds(abase, 640)])
    pltpu.sync_copy(zs, s_a.at[pl.ds(abase, 640)])
    plsc.subcore_barrier()

    ebase = sid * _NCH

    for rows_r, cols_r, sim_r, s_t in (
            (drows, dcols, dsim, s_d),
            (arows, acols, asim, s_a)):
        pltpu.sync_copy(rows_r.at[pl.ds(ebase, _NCH)], rows_t)
        pltpu.sync_copy(sim_r.at[pl.ds(ebase, _NCH)], vals_t)

        def _expb(i, _):
            for k in range(8):
                sl = pl.ds(16 * k, 16)
                vals_t[i, sl] = jnp.exp(vals_t[i, sl])
            return 0
        lax.fori_loop(0, _NCH, _expb, 0)

        def _sadd(j, _, s_t=s_t):
            pltpu.sync_copy(vals_t.at[j], s_t.at[rows_t.at[j]], add=True)
            return 0
        lax.fori_loop(0, _NCH, _sadd, 0)
        plsc.subcore_barrier()

        def _chunk(j, _, cols_r=cols_r, s_t=s_t):
            pltpu.sync_copy(s_t.at[rows_t.at[j]], sbuf)
            for k in range(8):
                sl = pl.ds(16 * k, 16)
                vals_t[j, sl] = vals_t[j, sl] / sbuf[sl] * 0.25
            pltpu.sync_copy(cols_r.at[pl.ds(ebase + j, 1)], colb)
            for k in range(8):
                sl = pl.ds(16 * k, 16)
                colb[0, sl] = colb[0, sl] * 2 + cid
            pltpu.sync_copy(user2.at[colb.at[0]], rowbuf)

            def _scale(g, _2):
                cv = vals_t[j, pl.ds(16 * g, 16)]
                for l in range(16):
                    c = cv[l]
                    r = 16 * g + l
                    for k in range(8):
                        sl = pl.ds(16 * k, 16)
                        rowbuf[r, sl] = rowbuf[r, sl] * c
                return 0
            lax.fori_loop(0, _CH // 16, _scale, 0)
            pltpu.sync_copy(rowbuf, acc.at[rows_t.at[j]], add=True)
            return 0
        lax.fori_loop(0, _NCH, _chunk, 0)

    obase = sid * 256
    pltpu.sync_copy(batch.at[sid], bidx)
    for jj in range(2):
        for k in range(8):
            sl = pl.ds(16 * k, 16)
            gbuf[0, sl] = bidx[jj, sl] * 2 + cid
        pltpu.sync_copy(user2.at[gbuf.at[0]], rowbuf)
        pltpu.sync_copy(rowbuf, emb_o.at[cid, pl.ds(obase + jj * _CH, _CH)])

        def _half(r, _2):
            for k in range(8):
                sl = pl.ds(16 * k, 16)
                rowbuf[r, sl] = rowbuf[r, sl] * 0.5
            return 0
        lax.fori_loop(0, _CH, _half, 0)
        pltpu.sync_copy(rowbuf, acc.at[bidx.at[jj]], add=True)

        pltpu.sync_copy(ego2.at[gbuf.at[0]], rowbuf)
        pltpu.sync_copy(rowbuf, ego_o.at[cid, pl.ds(obase + jj * _CH, _CH)])
    plsc.subcore_barrier()

    for jj in range(2):
        pltpu.sync_copy(acc.at[bidx.at[jj]], rowbuf)
        pltpu.sync_copy(rowbuf, uu_o.at[cid, pl.ds(obase + jj * _CH, _CH)])


def _mm_body(ego_r, emb_r, uu_r, w_r, b_r, o_r):
    a = jnp.dot(ego_r[0], w_r[pl.ds(0, _H), :], preferred_element_type=jnp.float32)
    a = a + jnp.dot(ego_r[1], w_r[pl.ds(_H, _H), :], preferred_element_type=jnp.float32)
    a = a + jnp.dot(emb_r[0], w_r[pl.ds(2 * _H, _H), :], preferred_element_type=jnp.float32)
    a = a + jnp.dot(emb_r[1], w_r[pl.ds(3 * _H, _H), :], preferred_element_type=jnp.float32)
    a = a + jnp.dot(uu_r[0], w_r[pl.ds(4 * _H, _H), :], preferred_element_type=jnp.float32)
    a = a + jnp.dot(uu_r[1], w_r[pl.ds(5 * _H, _H), :], preferred_element_type=jnp.float32)
    o_r[...] = a + b_r[...]


@jax.jit
def kernel(user_emb, user_emb_ego, dele_sim, add_sim, W_map, b_map,
           dele_indices, add_indices, batch_user):
    i32 = jnp.int32
    f32 = jnp.float32
    pad = _EP - _E

    def _prep_idx(x, val):
        x = x.astype(i32)
        return jnp.concatenate(
            [x, jnp.full((pad,), val, i32)]).reshape(_EP // _CH, _CH)

    def _prep_sim(x):
        return jnp.concatenate(
            [x.astype(f32), jnp.zeros((pad,), f32)]).reshape(_EP // _CH, _CH)

    drows2 = _prep_idx(dele_indices[0], _PADROW)
    dcols2 = _prep_idx(dele_indices[1], 0)
    arows2 = _prep_idx(add_indices[0], _PADROW)
    acols2 = _prep_idx(add_indices[1], 0)
    dsim2 = _prep_sim(dele_sim)
    asim2 = _prep_sim(add_sim)
    batch2 = batch_user.astype(i32).reshape(_NT, 2, _CH)
    user2 = user_emb.reshape(2 * _N, _H)
    ego2 = user_emb_ego.reshape(2 * _N, _H)

    mesh = plsc.VectorSubcoreMesh(core_axis_name="c", subcore_axis_name="s")
    sc = pl.kernel(
        _sc_body,
        out_type=[jax.ShapeDtypeStruct((2, _B, _H), f32)] * 3,
        mesh=mesh,
        scratch_types=[
            pltpu.VMEM_SHARED((_N2, _H), f32),
            pltpu.VMEM_SHARED((_N2,), f32),
            pltpu.VMEM_SHARED((_N2,), f32),
            pltpu.VMEM((_NCH, _CH), i32),
            pltpu.VMEM((_NCH, _CH), f32),
            pltpu.VMEM((1, _CH), i32),
            pltpu.VMEM((1, _CH), i32),
            pltpu.VMEM((_CH,), f32),
            pltpu.VMEM((_CH, _H), f32),
            pltpu.VMEM((640,), f32),
            pltpu.VMEM((2, _CH), i32),
        ],
    )
    uu3, emb3, ego3 = sc(user2, ego2, drows2, dcols2, dsim2,
                         arows2, acols2, asim2, batch2)

    blk = 512
    out = pl.pallas_call(
        _mm_body,
        grid=(_B // blk,),
        in_specs=[
            pl.BlockSpec((2, blk, _H), lambda i: (0, i, 0)),
            pl.BlockSpec((2, blk, _H), lambda i: (0, i, 0)),
            pl.BlockSpec((2, blk, _H), lambda i: (0, i, 0)),
            pl.BlockSpec((3 * _D, _D), lambda i: (0, 0)),
            pl.BlockSpec((1, _D), lambda i: (0, 0)),
        ],
        out_specs=pl.BlockSpec((blk, _D), lambda i: (i, 0)),
        out_shape=jax.ShapeDtypeStruct((_B, _D), f32),
    )(ego3, emb3, uu3, W_map, b_map.reshape(1, _D))
    return out
